# baseline (speedup 1.0000x reference)
import functools

import numpy as np
import jax
import jax.numpy as jnp
from jax import lax
from jax.experimental import pallas as pl
from jax.experimental.pallas import tpu as pltpu
from jax.experimental.pallas import tpu_sc as plsc

_HASH = 16
_CELLS = 1 << _HASH
_NC, _NS, _L = 2, 16, 16
_NW = _NC * _NS
_SPR = 128 // _HASH

def _ids_body(xt_ref, o_ref):
    blk = xt_ref.shape[1]
    k = lax.broadcasted_iota(jnp.int32, (_HASH, 1), 0)
    raw = jax.lax.bitcast_convert_type(xt_ref[...], jnp.int32)
    bits = jax.lax.shift_right_logical(raw, 29) & 1
    ids = jnp.sum(bits << k, axis=0)
    o_ref[...] = ids.reshape(blk // 128, 128)


def _compute_ids(xt, start_blk, nblk):
    blk = 65536
    return pl.pallas_call(
        _ids_body,
        grid=(nblk,),
        in_specs=[pl.BlockSpec((_HASH, blk), lambda i: (0, i + start_blk))],
        out_specs=pl.BlockSpec((blk // 128, 128), lambda i: (i, 0)),
        out_shape=jax.ShapeDtypeStruct((nblk * blk // 128, 128), jnp.int32),
    )(xt)


def _hist_body(ids_hbm, out_hbm, idx_v, ones_v, stage_v, hist_sh, sem):
    c = lax.axis_index("c")
    s = lax.axis_index("s")
    wid = c * _NS + s
    nrows = ids_hbm.shape[0] // _NW
    slc = _CELLS // _NS

    @plsc.parallel_loop(0, slc // _L, 1, unroll=8)
    def _zero(i):
        stage_v[pl.ds(i * _L, _L)] = jnp.zeros((_L,), jnp.float32)

    pltpu.sync_copy(stage_v, hist_sh.at[pl.ds(s * slc, slc)])

    @plsc.parallel_loop(0, 128 // _L, 1, unroll=8)
    def _one(i):
        ones_v[pl.ds(i * _L, _L)] = jnp.ones((_L,), jnp.float32)

    pltpu.sync_copy(ids_hbm.at[pl.ds(wid * nrows, nrows)], idx_v)
    plsc.subcore_barrier()

    k = 16

    def _grp(g, _):
        descs = []
        for u in range(k):
            descs.append(pltpu.async_copy(
                ones_v, hist_sh.at[idx_v.at[g * k + u]], sem, add=True))
        for d in descs:
            d.wait()
        return _

    lax.fori_loop(0, nrows // k, _grp, None)
    plsc.subcore_barrier()

    pltpu.sync_copy(hist_sh.at[pl.ds(s * slc, slc)],
                    out_hbm.at[c, pl.ds(s * slc, slc)])


_SC_PARAMS = pltpu.CompilerParams(needs_layout_passes=False)


def _hist(ids2d):
    nrows = ids2d.shape[0] // _NW
    mesh = plsc.VectorSubcoreMesh(core_axis_name="c", subcore_axis_name="s")
    return pl.kernel(
        _hist_body,
        out_type=jax.ShapeDtypeStruct((_NC, _CELLS), jnp.float32),
        mesh=mesh,
        compiler_params=_SC_PARAMS,
        scratch_types=[
            pltpu.VMEM((nrows, 128), jnp.int32),
            pltpu.VMEM((128,), jnp.float32),
            pltpu.VMEM((_CELLS // _NS,), jnp.float32),
            pltpu.VMEM_SHARED((_CELLS,), jnp.float32),
            pltpu.SemaphoreType.DMA,
        ],
    )(ids2d)


def _tab_body(*refs):
    (*p_refs, cs_ref, o_ref) = refs
    tot = cs_ref[...]
    for p in p_refs:
        tot = tot + p[0] + p[1]
    o_ref[...] = jax.lax.rsqrt(jnp.maximum(tot, 1.0))


def _table(parts, counts_state):
    return pl.pallas_call(
        _tab_body,
        out_shape=jax.ShapeDtypeStruct((_CELLS,), jnp.float32),
    )(*parts, counts_state)


def _gather_body(*refs):
    (*ids_list, rtab_hbm, out_hbm, tbl_v, idx_v, res_v) = refs
    c = lax.axis_index("c")
    s = lax.axis_index("s")
    wid = c * _NS + s
    qrows = ids_list[0].shape[0] // _NW

    pltpu.sync_copy(rtab_hbm, tbl_v)

    for h, ids_h in enumerate(ids_list):
        base = wid * qrows
        pltpu.sync_copy(ids_h.at[pl.ds(base, qrows)], idx_v)

        @plsc.parallel_loop(0, qrows, 1, unroll=4)
        def _row(r):
            for k in range(128 // _L):
                vidx = idx_v[r, pl.ds(k * _L, _L)]
                res_v[r, pl.ds(k * _L, _L)] = plsc.load_gather(
                    tbl_v, [vidx])

        pltpu.sync_copy(
            res_v, out_hbm.at[pl.ds(h * ids_list[0].shape[0] + base, qrows)])


def _gather(ids_list, rtab):
    qrows = ids_list[0].shape[0] // _NW
    nch = len(ids_list)
    mesh = plsc.VectorSubcoreMesh(core_axis_name="c", subcore_axis_name="s")
    return pl.kernel(
        _gather_body,
        out_type=jax.ShapeDtypeStruct(
            (nch * ids_list[0].shape[0], 128), jnp.float32),
        mesh=mesh,
        compiler_params=_SC_PARAMS,
        scratch_types=[
            pltpu.VMEM((_CELLS,), jnp.float32),
            pltpu.VMEM((qrows, 128), jnp.int32),
            pltpu.VMEM((qrows, 128), jnp.float32),
        ],
    )(*ids_list, rtab)


_NCHUNK = 4


def kernel(cells, counts_state):
    n = cells.shape[0]
    xt = cells.T
    nblk = (n // 65536) // _NCHUNK
    ids_list, parts = [], []
    for ch in range(_NCHUNK):
        ids_ch = _compute_ids(xt, ch * nblk, nblk)
        ids_list.append(ids_ch)
        parts.append(_hist(ids_ch))
    rtab = _table(parts, counts_state)
    out2d = _gather(ids_list, rtab)
    return out2d.reshape(n)


# speedup vs baseline: 1.1321x; 1.1321x over previous
import functools

import numpy as np
import jax
import jax.numpy as jnp
from jax import lax
from jax.experimental import pallas as pl
from jax.experimental.pallas import tpu as pltpu
from jax.experimental.pallas import tpu_sc as plsc

_HASH = 16
_CELLS = 1 << _HASH
_NC, _NS, _L = 2, 16, 16
_NW = _NC * _NS
_SPR = 128 // _HASH

def _ids_body(xt_ref, o_ref):
    blk = xt_ref.shape[1]
    k = lax.broadcasted_iota(jnp.int32, (_HASH, 1), 0)
    raw = jax.lax.bitcast_convert_type(xt_ref[...], jnp.int32)
    bits = jax.lax.shift_right_logical(raw, 29) & 1
    ids = jnp.sum(bits << k, axis=0)
    o_ref[...] = ids.reshape(blk // 128, 128)


def _compute_ids(xt, start_blk, nblk):
    blk = 65536
    return pl.pallas_call(
        _ids_body,
        grid=(nblk,),
        in_specs=[pl.BlockSpec((_HASH, blk), lambda i: (0, i + start_blk))],
        out_specs=pl.BlockSpec((blk // 128, 128), lambda i: (i, 0)),
        out_shape=jax.ShapeDtypeStruct((nblk * blk // 128, 128), jnp.int32),
    )(xt)


def _hist_body(ids_hbm, out_hbm, idx_v, ones_v, stage_v, hist_sh, sem):
    c = lax.axis_index("c")
    s = lax.axis_index("s")
    wid = c * _NS + s
    nrows = ids_hbm.shape[0] // _NW
    slc = _CELLS // _NS

    @plsc.parallel_loop(0, slc // _L, 1, unroll=8)
    def _zero(i):
        stage_v[pl.ds(i * _L, _L)] = jnp.zeros((_L,), jnp.float32)

    pltpu.sync_copy(stage_v, hist_sh.at[pl.ds(s * slc, slc)])

    @plsc.parallel_loop(0, 128 // _L, 1, unroll=8)
    def _one(i):
        ones_v[pl.ds(i * _L, _L)] = jnp.ones((_L,), jnp.float32)

    pltpu.sync_copy(ids_hbm.at[pl.ds(wid * nrows, nrows)], idx_v)
    plsc.subcore_barrier()

    k = 16

    def _grp(g, _):
        descs = []
        for u in range(k):
            descs.append(pltpu.async_copy(
                ones_v, hist_sh.at[idx_v.at[g * k + u]], sem, add=True))
        for d in descs:
            d.wait()
        return _

    lax.fori_loop(0, nrows // k, _grp, None)
    plsc.subcore_barrier()

    pltpu.sync_copy(hist_sh.at[pl.ds(s * slc, slc)],
                    out_hbm.at[c, pl.ds(s * slc, slc)])


_SC_PARAMS = pltpu.CompilerParams(needs_layout_passes=False)


def _hist(ids2d):
    nrows = ids2d.shape[0] // _NW
    mesh = plsc.VectorSubcoreMesh(core_axis_name="c", subcore_axis_name="s")
    return pl.kernel(
        _hist_body,
        out_type=jax.ShapeDtypeStruct((_NC, _CELLS), jnp.float32),
        mesh=mesh,
        compiler_params=_SC_PARAMS,
        scratch_types=[
            pltpu.VMEM((nrows, 128), jnp.int32),
            pltpu.VMEM((128,), jnp.float32),
            pltpu.VMEM((_CELLS // _NS,), jnp.float32),
            pltpu.VMEM_SHARED((_CELLS,), jnp.float32),
            pltpu.SemaphoreType.DMA,
        ],
    )(ids2d)


def _tab_body(*refs):
    (*p_refs, cs_ref, o_ref) = refs
    tot = cs_ref[...]
    for p in p_refs:
        tot = tot + p[0] + p[1]
    o_ref[...] = jax.lax.rsqrt(jnp.maximum(tot, 1.0))


def _table(parts, counts_state):
    return pl.pallas_call(
        _tab_body,
        out_shape=jax.ShapeDtypeStruct((_CELLS,), jnp.float32),
    )(*parts, counts_state)


def _gather_body(*refs):
    (*ids_list, rtab_hbm, out_hbm, tbl_v, idx_v, res_v) = refs
    c = lax.axis_index("c")
    s = lax.axis_index("s")
    wid = c * _NS + s
    qrows = ids_list[0].shape[0] // _NW

    pltpu.sync_copy(rtab_hbm, tbl_v)

    for h, ids_h in enumerate(ids_list):
        base = wid * qrows
        pltpu.sync_copy(ids_h.at[pl.ds(base, qrows)], idx_v)

        @plsc.parallel_loop(0, qrows, 1, unroll=4)
        def _row(r):
            for k in range(128 // _L):
                vidx = idx_v[r, pl.ds(k * _L, _L)]
                res_v[r, pl.ds(k * _L, _L)] = plsc.load_gather(
                    tbl_v, [vidx])

        pltpu.sync_copy(
            res_v, out_hbm.at[pl.ds(h * ids_list[0].shape[0] + base, qrows)])


def _gather(ids_list, rtab):
    qrows = ids_list[0].shape[0] // _NW
    nch = len(ids_list)
    mesh = plsc.VectorSubcoreMesh(core_axis_name="c", subcore_axis_name="s")
    return pl.kernel(
        _gather_body,
        out_type=jax.ShapeDtypeStruct(
            (nch * ids_list[0].shape[0], 128), jnp.float32),
        mesh=mesh,
        compiler_params=_SC_PARAMS,
        scratch_types=[
            pltpu.VMEM((_CELLS,), jnp.float32),
            pltpu.VMEM((qrows, 128), jnp.int32),
            pltpu.VMEM((qrows, 128), jnp.float32),
        ],
    )(*ids_list, rtab)


_NCHUNK = 2


def kernel(cells, counts_state):
    n = cells.shape[0]
    xt = cells.T
    nblk = (n // 65536) // _NCHUNK
    ids_list, parts = [], []
    for ch in range(_NCHUNK):
        ids_ch = _compute_ids(xt, ch * nblk, nblk)
        ids_list.append(ids_ch)
        parts.append(_hist(ids_ch))
    rtab = _table(parts, counts_state)
    out2d = _gather(ids_list, rtab)
    return out2d.reshape(n)


# speedup vs baseline: 1.1388x; 1.0059x over previous
import functools

import numpy as np
import jax
import jax.numpy as jnp
from jax import lax
from jax.experimental import pallas as pl
from jax.experimental.pallas import tpu as pltpu
from jax.experimental.pallas import tpu_sc as plsc

_HASH = 16
_CELLS = 1 << _HASH
_NC, _NS, _L = 2, 16, 16
_NW = _NC * _NS
_SPR = 128 // _HASH

def _ids_body(xt_ref, o_ref):
    blk = xt_ref.shape[1]
    k = lax.broadcasted_iota(jnp.int32, (_HASH, 1), 0)
    raw = jax.lax.bitcast_convert_type(xt_ref[...], jnp.int32)
    bits = jax.lax.shift_right_logical(raw, 29) & 1
    ids = jnp.sum(bits << k, axis=0)
    o_ref[...] = ids.reshape(blk // 128, 128)


def _compute_ids(xt, start_blk, nblk):
    blk = 65536
    return pl.pallas_call(
        _ids_body,
        grid=(nblk,),
        in_specs=[pl.BlockSpec((_HASH, blk), lambda i: (0, i + start_blk))],
        out_specs=pl.BlockSpec((blk // 128, 128), lambda i: (i, 0)),
        out_shape=jax.ShapeDtypeStruct((nblk * blk // 128, 128), jnp.int32),
    )(xt)


def _hist_body(ids_hbm, out_hbm, idx_v, ones_v, stage_v, hist_sh, sem):
    c = lax.axis_index("c")
    s = lax.axis_index("s")
    wid = c * _NS + s
    nrows = ids_hbm.shape[0] // _NW
    slc = _CELLS // _NS

    @plsc.parallel_loop(0, slc // _L, 1, unroll=8)
    def _zero(i):
        stage_v[pl.ds(i * _L, _L)] = jnp.zeros((_L,), jnp.float32)

    pltpu.sync_copy(stage_v, hist_sh.at[pl.ds(s * slc, slc)])

    @plsc.parallel_loop(0, 128 // _L, 1, unroll=8)
    def _one(i):
        ones_v[pl.ds(i * _L, _L)] = jnp.ones((_L,), jnp.float32)

    pltpu.sync_copy(ids_hbm.at[pl.ds(wid * nrows, nrows)], idx_v)
    plsc.subcore_barrier()

    k = 16

    def _grp(g, _):
        descs = []
        for u in range(k):
            descs.append(pltpu.async_copy(
                ones_v, hist_sh.at[idx_v.at[g * k + u]], sem, add=True))
        for d in descs:
            d.wait()
        return _

    lax.fori_loop(0, nrows // k, _grp, None)
    plsc.subcore_barrier()

    pltpu.sync_copy(hist_sh.at[pl.ds(s * slc, slc)],
                    out_hbm.at[c, pl.ds(s * slc, slc)])


_SC_PARAMS = pltpu.CompilerParams(needs_layout_passes=False)


def _hist(ids2d):
    nrows = ids2d.shape[0] // _NW
    mesh = plsc.VectorSubcoreMesh(core_axis_name="c", subcore_axis_name="s")
    return pl.kernel(
        _hist_body,
        out_type=jax.ShapeDtypeStruct((_NC, _CELLS), jnp.float32),
        mesh=mesh,
        compiler_params=_SC_PARAMS,
        scratch_types=[
            pltpu.VMEM((nrows, 128), jnp.int32),
            pltpu.VMEM((128,), jnp.float32),
            pltpu.VMEM((_CELLS // _NS,), jnp.float32),
            pltpu.VMEM_SHARED((_CELLS,), jnp.float32),
            pltpu.SemaphoreType.DMA,
        ],
    )(ids2d)


def _tab_body(*refs):
    (*p_refs, cs_ref, o_ref) = refs
    tot = cs_ref[...]
    for p in p_refs:
        tot = tot + p[0] + p[1]
    o_ref[...] = jax.lax.rsqrt(jnp.maximum(tot, 1.0))


def _table(parts, counts_state):
    return pl.pallas_call(
        _tab_body,
        out_shape=jax.ShapeDtypeStruct((_CELLS,), jnp.float32),
    )(*parts, counts_state)


def _gather_body(*refs):
    (*ids_list, rtab_hbm, out_hbm, tbl_v, idx_v, res_v, sem_t, sem_i,
     sem_o) = refs
    c = lax.axis_index("c")
    s = lax.axis_index("s")
    wid = c * _NS + s

    tbl_cp = pltpu.async_copy(rtab_hbm, tbl_v, sem_t)
    row_off = 0
    qr0 = ids_list[0].shape[0] // _NW
    first_cp = pltpu.async_copy(
        ids_list[0].at[pl.ds(wid * qr0, qr0)], idx_v.at[pl.ds(0, qr0)], sem_i)
    tbl_cp.wait()
    first_cp.wait()

    out_cp = None
    for h, ids_h in enumerate(ids_list):
        qrows = ids_h.shape[0] // _NW
        base = wid * qrows

        @plsc.parallel_loop(0, qrows, 1, unroll=4)
        def _row(r):
            for k in range(128 // _L):
                vidx = idx_v[r, pl.ds(k * _L, _L)]
                res_v[r, pl.ds(k * _L, _L)] = plsc.load_gather(
                    tbl_v, [vidx])

        if out_cp is not None:
            out_cp.wait()
        out_cp = pltpu.async_copy(
            res_v.at[pl.ds(0, qrows)],
            out_hbm.at[pl.ds(row_off + base, qrows)], sem_o)
        row_off += ids_h.shape[0]
        if h + 1 < len(ids_list):
            qn = ids_list[h + 1].shape[0] // _NW
            pltpu.async_copy(
                ids_list[h + 1].at[pl.ds(wid * qn, qn)],
                idx_v.at[pl.ds(0, qn)], sem_i).wait()
    out_cp.wait()


def _gather(ids_list, rtab):
    qmax = max(x.shape[0] for x in ids_list) // _NW
    nrows_total = sum(x.shape[0] for x in ids_list)
    mesh = plsc.VectorSubcoreMesh(core_axis_name="c", subcore_axis_name="s")
    return pl.kernel(
        _gather_body,
        out_type=jax.ShapeDtypeStruct((nrows_total, 128), jnp.float32),
        mesh=mesh,
        compiler_params=_SC_PARAMS,
        scratch_types=[
            pltpu.VMEM((_CELLS,), jnp.float32),
            pltpu.VMEM((qmax, 128), jnp.int32),
            pltpu.VMEM((qmax, 128), jnp.float32),
            pltpu.SemaphoreType.DMA,
            pltpu.SemaphoreType.DMA,
            pltpu.SemaphoreType.DMA,
        ],
    )(*ids_list, rtab)


_SPLIT_BLKS = (9, 7)


def kernel(cells, counts_state):
    n = cells.shape[0]
    xt = cells.T
    ids_list, parts = [], []
    start = 0
    for nblk in _SPLIT_BLKS:
        ids_ch = _compute_ids(xt, start, nblk)
        start += nblk
        ids_list.append(ids_ch)
        parts.append(_hist(ids_ch))
    rtab = _table(parts, counts_state)
    out2d = _gather(ids_list, rtab)
    return out2d.reshape(n)
